# baseline (speedup 1.0000x reference)
import functools

import jax
import jax.numpy as jnp
from jax import lax
from jax.experimental import pallas as pl
from jax.experimental.pallas import tpu as pltpu
from jax.experimental.pallas import tpu_sc as plsc

NC, NS = 2, 16
NW = NC * NS
BB = 16384
BP = BB // 2
F = 26
D = 128
NT = 10
NP = NT * NT
PPW = BP // NW
CHP = 128
NH = PPW // CHP
NG = F * NH


def _sc_body(idx_hbm, table_hbm, out_hbm, idx_v, rows_v, tt_v, pair_sh,
             sem_s, sem_g, sem_w):
    sid = lax.axis_index("s")
    wid = sid * NC + lax.axis_index("c")
    p0 = wid * PPW
    pltpu.sync_copy(idx_hbm.at[:, pl.ds(p0, PPW)], idx_v)

    def build_pairs():
        pltpu.sync_copy(table_hbm, tt_v)
        for i in range(NT):
            def fill_row(i=i):
                for j in range(NT):
                    p = i * NT + j
                    pltpu.async_copy(tt_v.at[i], pair_sh.at[p, 0], sem_s)
                    pltpu.async_copy(tt_v.at[j], pair_sh.at[p, 1], sem_s)

            pl.when(sid == i)(fill_row)

        def drain():
            for _ in range(2 * NT):
                pltpu.make_async_copy(tt_v.at[0], pair_sh.at[0, 0],
                                      sem_s).wait()

        pl.when(sid < NT)(drain)

    build_pairs()
    plsc.subcore_barrier()

    def fire_gather(buf, g):
        f = lax.div(g, NH)
        half = lax.rem(g, NH)
        pltpu.async_copy(
            pair_sh.at[idx_v.at[f, pl.ds(half * CHP, CHP)]],
            rows_v.at[buf], sem_g)

    def wait_gather():
        pltpu.make_async_copy(
            pair_sh.at[idx_v.at[0, pl.ds(0, CHP)]],
            rows_v.at[0], sem_g).wait()

    def start_write(buf, g):
        f = lax.div(g, NH)
        half = lax.rem(g, NH)
        pltpu.async_copy(
            rows_v.at[buf], out_hbm.at[f, pl.ds(p0 + half * CHP, CHP)], sem_w)

    def wait_write():
        pltpu.make_async_copy(
            rows_v.at[0], out_hbm.at[0, pl.ds(p0, CHP)], sem_w).wait()

    fire_gather(0, 0)
    fire_gather(1, 1)

    def body(g, carry):
        wait_gather()
        pl.when(g > 0)(wait_write)
        start_write(lax.rem(g, 3), g)

        def prefetch():
            fire_gather(lax.rem(g + 2, 3), g + 2)

        pl.when(g < NG - 2)(prefetch)
        return carry

    lax.fori_loop(0, NG, body, 0)
    wait_write()


@jax.jit
def kernel(indices, table):
    idx_t = jnp.transpose(indices.astype(jnp.int32))
    pidx = idx_t[:, 0::2] * NT + idx_t[:, 1::2]
    mesh = plsc.VectorSubcoreMesh(core_axis_name="c", subcore_axis_name="s")
    k = functools.partial(
        pl.kernel,
        out_type=jax.ShapeDtypeStruct((F, BP, 2, D), jnp.float32),
        mesh=mesh,
        scratch_types=[
            pltpu.VMEM((F, PPW), jnp.int32),
            pltpu.VMEM((3, CHP, 2, D), jnp.float32),
            pltpu.VMEM((NT, D), jnp.float32),
            pltpu.VMEM_SHARED((NP, 2, D), jnp.float32),
            pltpu.SemaphoreType.DMA,
            pltpu.SemaphoreType.DMA,
            pltpu.SemaphoreType.DMA,
        ],
    )(_sc_body)
    out_t = k(pidx, table)
    out_t = out_t.reshape(F, BB, D)
    return jnp.transpose(out_t, (1, 0, 2))


# speedup vs baseline: 1.8904x; 1.8904x over previous
import functools

import jax
import jax.numpy as jnp
from jax import lax
from jax.experimental import pallas as pl
from jax.experimental.pallas import tpu as pltpu
from jax.experimental.pallas import tpu_sc as plsc

NC, NS = 2, 16
NW = NC * NS
BB = 16384
F = 26
D = 128
BPW = BB // NW
CHB = 256
NH = BPW // CHB
CH = 128
NGPB = CHB // CH
NG = F * NH


def _sc_body(idx_hbm, table_hbm, out_hbm, idx_v, rows_v, table_sh, sem_g, sem_w):
    sid = lax.axis_index("s")
    wid = sid * NC + lax.axis_index("c")
    b0 = wid * BPW
    pltpu.sync_copy(idx_hbm.at[:, pl.ds(b0, BPW)], idx_v)

    def stage_table():
        pltpu.sync_copy(table_hbm, table_sh)

    pl.when(sid == 0)(stage_table)
    plsc.subcore_barrier()

    def fire_gather(buf, g):
        f = lax.div(g, NH)
        half = lax.rem(g, NH)
        for k in range(NGPB):
            pltpu.async_copy(
                table_sh.at[idx_v.at[f, pl.ds(half * CHB + k * CH, CH)]],
                rows_v.at[buf, pl.ds(k * CH, CH)], sem_g)

    def wait_gather():
        for k in range(NGPB):
            pltpu.make_async_copy(
                table_sh.at[idx_v.at[0, pl.ds(0, CH)]],
                rows_v.at[0, pl.ds(0, CH)], sem_g).wait()

    def start_write(buf, g):
        f = lax.div(g, NH)
        half = lax.rem(g, NH)
        pltpu.async_copy(
            rows_v.at[buf], out_hbm.at[f, pl.ds(b0 + half * CHB, CHB)], sem_w)

    def wait_write():
        pltpu.make_async_copy(
            rows_v.at[0], out_hbm.at[0, pl.ds(b0, CHB)], sem_w).wait()

    fire_gather(0, 0)
    fire_gather(1, 1)

    def body(g, carry):
        wait_gather()
        start_write(lax.rem(g, 3), g)
        pl.when(g > 0)(wait_write)

        def prefetch():
            fire_gather(lax.rem(g + 2, 3), g + 2)

        pl.when(g < NG - 2)(prefetch)
        return carry

    lax.fori_loop(0, NG, body, 0)
    wait_write()


@jax.jit
def kernel(indices, table):
    idx_t = jnp.transpose(indices.astype(jnp.int32))
    mesh = plsc.VectorSubcoreMesh(core_axis_name="c", subcore_axis_name="s")
    k = functools.partial(
        pl.kernel,
        out_type=jax.ShapeDtypeStruct((F, BB, D), jnp.float32),
        mesh=mesh,
        scratch_types=[
            pltpu.VMEM((F, BPW), jnp.int32),
            pltpu.VMEM((3, CHB, D), jnp.float32),
            pltpu.VMEM_SHARED((10, D), jnp.float32),
            pltpu.SemaphoreType.DMA,
            pltpu.SemaphoreType.DMA,
        ],
    )(_sc_body)
    out_t = k(idx_t, table)
    return jnp.transpose(out_t, (1, 0, 2))
